# Initial kernel scaffold; baseline (speedup 1.0000x reference)
import jax
import jax.numpy as jnp
from jax.experimental import pallas as pl


def kernel(x, wr, br, e0_w, e0_b, e1_w, e1_b, e2_wd, e2_wp, e2_b, e3_ws, e3_bs, e4_w, e4_b, e5_w, e5_b, warmup_factor):
    raise NotImplementedError("write your pallas kernel here")


# speedup vs baseline: 2.5462x; 2.5462x over previous
import functools

import jax
import jax.numpy as jnp
from jax.experimental import pallas as pl
from jax.experimental.pallas import tpu as pltpu

B, C, H, W = 4, 192, 56, 56
E, K = 6, 2
HW = H * W


def _router_kernel(x_ref, wr_ref, br_ref, ws_ref, bs_ref, wf_ref,
                   wsel_ref, isel_ref, s_ref, total_ref):
    x = x_ref[...]
    pooled = jnp.sum(x, axis=(1, 2)) * (1.0 / HW)
    logits = jnp.clip(
        jnp.dot(pooled, wr_ref[...], preferred_element_type=jnp.float32)
        + br_ref[...][None, :], -10.0, 10.0)
    z = logits - jnp.max(logits, axis=1, keepdims=True)
    ez = jnp.exp(z)
    probs = jnp.clip(ez / jnp.sum(ez, axis=1, keepdims=True), 1e-06, 1.0)

    iota = jax.lax.broadcasted_iota(jnp.int32, (B, E), 1)
    m1 = jnp.max(probs, axis=1, keepdims=True)
    i1 = jnp.min(jnp.where(probs == m1, iota, E), axis=1)
    masked = jnp.where(iota == i1[:, None], -jnp.inf, probs)
    m2 = jnp.max(masked, axis=1, keepdims=True)
    i2 = jnp.min(jnp.where(masked == m2, iota, E), axis=1)
    p1 = m1[:, 0]
    p2 = m2[:, 0]
    denom = 1.0 / (p1 + p2 + 1e-08)
    wsel_ref[...] = jnp.concatenate(
        [(p1 * denom)[:, None], (p2 * denom)[:, None]], axis=1)
    isel_ref[...] = jnp.concatenate([i1[:, None], i2[:, None]], axis=1)

    s_ref[...] = jax.nn.sigmoid(
        jnp.dot(pooled, ws_ref[...], preferred_element_type=jnp.float32)
        + bs_ref[...][None, :])

    usage = jnp.mean(probs, axis=0)
    lb = jnp.sum((usage - 1.0 / E) ** 2)
    entropy = -jnp.mean(jnp.sum(probs * jnp.log(probs + 1e-10), axis=1))
    coef = 1e-05 + wf_ref[0, 0] * (0.0005 - 1e-05)
    total_ref[...] = jnp.full((1, 1), lb * coef - entropy * 0.001)


def _expert_kernel(isel_ref, wsel_ref,
                   x_ref, taps0_ref, taps4_ref, m1_ref, m5_ref,
                   wd_ref, m2p_ref, bias_ref, s_ref, out_ref):
    b = pl.program_id(0)
    k = pl.program_id(1)
    e = isel_ref[b * K + k]
    w = wsel_ref[b * K + k]

    xp = x_ref[0]
    xc = xp[1:1 + H, 1:1 + W, :]

    @pl.when(k == 0)
    def _init():
        out_ref[0] = jnp.zeros((H, W, C), jnp.float32)

    def conv3x3_branch(eid, taps_ref):
        @pl.when(e == eid)
        def _():
            y = jnp.zeros((HW, C), jnp.float32)
            for t in range(9):
                dh, dw = t // 3, t % 3
                xs = xp[dh:dh + H, dw:dw + W, :].reshape(HW, C)
                y = y + jnp.dot(xs, taps_ref[t],
                                preferred_element_type=jnp.float32)
            y = jnp.maximum(y + bias_ref[eid][None, :], 0.0).reshape(H, W, C)
            out_ref[0] += w * (xc + y)

    def conv1x1_branch(eid, m_ref):
        @pl.when(e == eid)
        def _():
            y = jnp.dot(xc.reshape(HW, C), m_ref[...],
                        preferred_element_type=jnp.float32)
            y = jnp.maximum(y + bias_ref[eid][None, :], 0.0).reshape(H, W, C)
            out_ref[0] += w * (xc + y)

    conv3x3_branch(0, taps0_ref)
    conv1x1_branch(1, m1_ref)

    @pl.when(e == 2)
    def _edge():
        yd = jnp.zeros((H, W, C), jnp.float32)
        for t in range(9):
            dh, dw = t // 3, t % 3
            yd = yd + xp[dh:dh + H, dw:dw + W, :] * wd_ref[t][None, None, :]
        y = jnp.dot(yd.reshape(HW, C), m2p_ref[...],
                    preferred_element_type=jnp.float32)
        y = jnp.maximum(y + bias_ref[2][None, :], 0.0).reshape(H, W, C)
        out_ref[0] += w * (xc + y)

    @pl.when(e == 3)
    def _contrast():
        out_ref[0] += xc * (w * s_ref[0, 0])[None, None, :]

    conv3x3_branch(4, taps4_ref)
    conv1x1_branch(5, m5_ref)


def kernel(x, wr, br, e0_w, e0_b, e1_w, e1_b, e2_wd, e2_wp, e2_b,
           e3_ws, e3_bs, e4_w, e4_b, e5_w, e5_b, warmup_factor=1.0):
    x_pad = jnp.pad(jnp.transpose(x, (0, 2, 3, 1)),
                    ((0, 0), (1, 1), (1, 1), (0, 0)))
    wf = jnp.asarray(warmup_factor, jnp.float32).reshape(1, 1)

    wsel, isel, s, total = pl.pallas_call(
        _router_kernel,
        out_shape=(
            jax.ShapeDtypeStruct((B, K), jnp.float32),
            jax.ShapeDtypeStruct((B, K), jnp.int32),
            jax.ShapeDtypeStruct((B, C), jnp.float32),
            jax.ShapeDtypeStruct((1, 1), jnp.float32),
        ),
    )(x_pad, wr, br, e3_ws, e3_bs, wf)

    taps0 = jnp.transpose(e0_w, (2, 3, 1, 0)).reshape(9, C, C)
    taps4 = jnp.transpose(e4_w, (2, 3, 1, 0)).reshape(9, C, C)
    m1 = e1_w[:, :, 0, 0].T
    m5 = e5_w[:, :, 0, 0].T
    m2p = e2_wp[:, :, 0, 0].T
    wd = jnp.transpose(e2_wd[:, 0], (1, 2, 0)).reshape(9, C)
    bias = jnp.stack([e0_b, e1_b, e2_b, e2_b * 0.0, e4_b, e5_b], axis=0)

    grid_spec = pltpu.PrefetchScalarGridSpec(
        num_scalar_prefetch=2,
        grid=(B, K),
        in_specs=[
            pl.BlockSpec((1, H + 2, W + 2, C), lambda b, k, *_: (b, 0, 0, 0)),
            pl.BlockSpec((9, C, C), lambda b, k, *_: (0, 0, 0)),
            pl.BlockSpec((9, C, C), lambda b, k, *_: (0, 0, 0)),
            pl.BlockSpec((C, C), lambda b, k, *_: (0, 0)),
            pl.BlockSpec((C, C), lambda b, k, *_: (0, 0)),
            pl.BlockSpec((9, C), lambda b, k, *_: (0, 0)),
            pl.BlockSpec((C, C), lambda b, k, *_: (0, 0)),
            pl.BlockSpec((E, C), lambda b, k, *_: (0, 0)),
            pl.BlockSpec((1, 1, C), lambda b, k, *_: (b, 0, 0)),
        ],
        out_specs=pl.BlockSpec((1, H, W, C), lambda b, k, *_: (b, 0, 0, 0)),
    )

    out_hwc = pl.pallas_call(
        _expert_kernel,
        grid_spec=grid_spec,
        out_shape=jax.ShapeDtypeStruct((B, H, W, C), jnp.float32),
    )(isel.reshape(B * K), wsel.reshape(B * K),
      x_pad, taps0, taps4, m1, m5, wd, m2p, bias, s.reshape(B, 1, C))

    return jnp.transpose(out_hwc, (0, 3, 1, 2)), total.reshape(())


# speedup vs baseline: 2.5912x; 1.0177x over previous
import functools

import jax
import jax.numpy as jnp
from jax.experimental import pallas as pl
from jax.experimental.pallas import tpu as pltpu

B, C, H, W = 4, 192, 56, 56
E, K = 6, 2
HW = H * W


def _router_kernel(x_ref, wr_ref, br_ref, ws_ref, bs_ref, wf_ref,
                   wsel_ref, isel_ref, s_ref, total_ref):
    x = x_ref[...]
    pooled = jnp.sum(x, axis=(1, 2)) * (1.0 / HW)
    logits = jnp.clip(
        jnp.dot(pooled, wr_ref[...], preferred_element_type=jnp.float32)
        + br_ref[...][None, :], -10.0, 10.0)
    z = logits - jnp.max(logits, axis=1, keepdims=True)
    ez = jnp.exp(z)
    probs = jnp.clip(ez / jnp.sum(ez, axis=1, keepdims=True), 1e-06, 1.0)

    iota = jax.lax.broadcasted_iota(jnp.int32, (B, E), 1)
    m1 = jnp.max(probs, axis=1, keepdims=True)
    i1 = jnp.min(jnp.where(probs == m1, iota, E), axis=1)
    masked = jnp.where(iota == i1[:, None], -jnp.inf, probs)
    m2 = jnp.max(masked, axis=1, keepdims=True)
    i2 = jnp.min(jnp.where(masked == m2, iota, E), axis=1)
    p1 = m1[:, 0]
    p2 = m2[:, 0]
    denom = 1.0 / (p1 + p2 + 1e-08)
    wsel_ref[...] = jnp.concatenate(
        [(p1 * denom)[:, None], (p2 * denom)[:, None]], axis=1)
    isel_ref[...] = jnp.concatenate([i1[:, None], i2[:, None]], axis=1)

    s_ref[...] = jax.nn.sigmoid(
        jnp.dot(pooled, ws_ref[...], preferred_element_type=jnp.float32)
        + bs_ref[...][None, :])

    usage = jnp.mean(probs, axis=0)
    lb = jnp.sum((usage - 1.0 / E) ** 2)
    entropy = -jnp.mean(jnp.sum(probs * jnp.log(probs + 1e-10), axis=1))
    coef = 1e-05 + wf_ref[0, 0] * (0.0005 - 1e-05)
    total_ref[...] = jnp.full((1, 1), lb * coef - entropy * 0.001)


def _expert_kernel(isel_ref, wsel_ref,
                   x_ref, taps0_ref, taps4_ref, m1_ref, m5_ref,
                   wd_ref, m2p_ref, bias_ref, s_ref, out_ref):
    b = pl.program_id(0)
    k = pl.program_id(1)
    e = isel_ref[b * K + k]
    w = wsel_ref[b * K + k]

    xp = x_ref[0]
    xc = xp[1:1 + H, 1:1 + W, :]
    xb = xp.astype(jnp.bfloat16)

    def emit(v):
        @pl.when(k == 0)
        def _():
            out_ref[0] = v

        @pl.when(k != 0)
        def _():
            out_ref[0] += v

    def conv3x3_branch(eid, taps_ref):
        @pl.when(e == eid)
        def _():
            y = jnp.zeros((HW, C), jnp.float32)
            for t in range(9):
                dh, dw = t // 3, t % 3
                xs = xb[dh:dh + H, dw:dw + W, :].reshape(HW, C)
                y = y + jnp.dot(xs, taps_ref[t],
                                preferred_element_type=jnp.float32)
            y = jnp.maximum(y + bias_ref[eid][None, :], 0.0).reshape(H, W, C)
            emit(w * (xc + y))

    def conv1x1_branch(eid, m_ref):
        @pl.when(e == eid)
        def _():
            y = jnp.dot(xb[1:1 + H, 1:1 + W, :].reshape(HW, C), m_ref[...],
                        preferred_element_type=jnp.float32)
            y = jnp.maximum(y + bias_ref[eid][None, :], 0.0).reshape(H, W, C)
            emit(w * (xc + y))

    conv3x3_branch(0, taps0_ref)
    conv1x1_branch(1, m1_ref)

    @pl.when(e == 2)
    def _edge():
        yd = jnp.zeros((H, W, C), jnp.float32)
        for t in range(9):
            dh, dw = t // 3, t % 3
            yd = yd + xp[dh:dh + H, dw:dw + W, :] * wd_ref[t][None, None, :]
        y = jnp.dot(yd.astype(jnp.bfloat16).reshape(HW, C), m2p_ref[...],
                    preferred_element_type=jnp.float32)
        y = jnp.maximum(y + bias_ref[2][None, :], 0.0).reshape(H, W, C)
        emit(w * (xc + y))

    @pl.when(e == 3)
    def _contrast():
        emit(xc * (w * s_ref[0, 0])[None, None, :])

    conv3x3_branch(4, taps4_ref)
    conv1x1_branch(5, m5_ref)


def kernel(x, wr, br, e0_w, e0_b, e1_w, e1_b, e2_wd, e2_wp, e2_b,
           e3_ws, e3_bs, e4_w, e4_b, e5_w, e5_b, warmup_factor=1.0):
    x_pad = jnp.pad(jnp.transpose(x, (0, 2, 3, 1)),
                    ((0, 0), (1, 1), (1, 1), (0, 0)))
    wf = jnp.asarray(warmup_factor, jnp.float32).reshape(1, 1)

    wsel, isel, s, total = pl.pallas_call(
        _router_kernel,
        out_shape=(
            jax.ShapeDtypeStruct((B, K), jnp.float32),
            jax.ShapeDtypeStruct((B, K), jnp.int32),
            jax.ShapeDtypeStruct((B, C), jnp.float32),
            jax.ShapeDtypeStruct((1, 1), jnp.float32),
        ),
    )(x_pad, wr, br, e3_ws, e3_bs, wf)

    bf = jnp.bfloat16
    taps0 = jnp.transpose(e0_w, (2, 3, 1, 0)).reshape(9, C, C).astype(bf)
    taps4 = jnp.transpose(e4_w, (2, 3, 1, 0)).reshape(9, C, C).astype(bf)
    m1 = e1_w[:, :, 0, 0].T.astype(bf)
    m5 = e5_w[:, :, 0, 0].T.astype(bf)
    m2p = e2_wp[:, :, 0, 0].T.astype(bf)
    wd = jnp.transpose(e2_wd[:, 0], (1, 2, 0)).reshape(9, C)
    bias = jnp.stack([e0_b, e1_b, e2_b, e2_b * 0.0, e4_b, e5_b], axis=0)

    grid_spec = pltpu.PrefetchScalarGridSpec(
        num_scalar_prefetch=2,
        grid=(B, K),
        in_specs=[
            pl.BlockSpec((1, H + 2, W + 2, C), lambda b, k, *_: (b, 0, 0, 0)),
            pl.BlockSpec((9, C, C), lambda b, k, *_: (0, 0, 0)),
            pl.BlockSpec((9, C, C), lambda b, k, *_: (0, 0, 0)),
            pl.BlockSpec((C, C), lambda b, k, *_: (0, 0)),
            pl.BlockSpec((C, C), lambda b, k, *_: (0, 0)),
            pl.BlockSpec((9, C), lambda b, k, *_: (0, 0)),
            pl.BlockSpec((C, C), lambda b, k, *_: (0, 0)),
            pl.BlockSpec((E, C), lambda b, k, *_: (0, 0)),
            pl.BlockSpec((1, 1, C), lambda b, k, *_: (b, 0, 0)),
        ],
        out_specs=pl.BlockSpec((1, H, W, C), lambda b, k, *_: (b, 0, 0, 0)),
    )

    out_hwc = pl.pallas_call(
        _expert_kernel,
        grid_spec=grid_spec,
        out_shape=jax.ShapeDtypeStruct((B, H, W, C), jnp.float32),
    )(isel.reshape(B * K), wsel.reshape(B * K),
      x_pad, taps0, taps4, m1, m5, wd, m2p, bias, s.reshape(B, 1, C))

    return jnp.transpose(out_hwc, (0, 3, 1, 2)), total.reshape(())
